# Initial kernel scaffold; baseline (speedup 1.0000x reference)
import jax
import jax.numpy as jnp
from jax.experimental import pallas as pl


def kernel(dist, dist_embedder_weight):
    raise NotImplementedError("write your pallas kernel here")


# speedup vs baseline: 1.5718x; 1.5718x over previous
import functools

import jax
import jax.numpy as jnp
from jax import lax
from jax.experimental import pallas as pl
from jax.experimental.pallas import tpu as pltpu
from jax.experimental.pallas import tpu_sc as plsc

VOCAB = 1000000
EMBED_DIM = 64
BATCH = 16384
HIST = 50
N_ROWS = BATCH * HIST

_INFO = plsc.get_sparse_core_info()
NC = _INFO.num_cores
NS = _INFO.num_subcores
NW = NC * NS

ROWS_PER_W = N_ROWS // NW
CHUNK = 128
N_CHUNKS = ROWS_PER_W // CHUNK

_mesh = plsc.VectorSubcoreMesh(core_axis_name="c", subcore_axis_name="s")


@functools.partial(
    pl.kernel,
    mesh=_mesh,
    out_type=jax.ShapeDtypeStruct((N_ROWS, EMBED_DIM), jnp.float32),
    compiler_params=pltpu.CompilerParams(use_tc_tiling_on_sc=False),
    scratch_types=[
        pltpu.VMEM((CHUNK,), jnp.int32),
        pltpu.VMEM((CHUNK, EMBED_DIM), jnp.float32),
        pltpu.SemaphoreType.DMA,
    ],
)
def _gather_kernel(idx_hbm, table_hbm, out_hbm, idx_v, rows_v, sem):
    wid = lax.axis_index("s") * NC + lax.axis_index("c")

    def body(i, carry):
        base = (wid * N_CHUNKS + i) * CHUNK
        pltpu.sync_copy(idx_hbm.at[pl.ds(base, CHUNK)], idx_v)
        pltpu.async_copy(table_hbm.at[idx_v], rows_v, sem).wait()
        pltpu.sync_copy(rows_v, out_hbm.at[pl.ds(base, CHUNK)])
        return carry

    lax.fori_loop(0, N_CHUNKS, body, 0)


def kernel(dist, dist_embedder_weight):
    idx_flat = dist.reshape(N_ROWS).astype(jnp.int32)
    out = _gather_kernel(idx_flat, dist_embedder_weight)
    return out.reshape(BATCH, HIST, EMBED_DIM)


# speedup vs baseline: 1.8646x; 1.1863x over previous
import functools

import jax
import jax.numpy as jnp
from jax import lax
from jax.experimental import pallas as pl
from jax.experimental.pallas import tpu as pltpu
from jax.experimental.pallas import tpu_sc as plsc

VOCAB = 1000000
EMBED_DIM = 64
BATCH = 16384
HIST = 50
N_ROWS = BATCH * HIST

_INFO = plsc.get_sparse_core_info()
NC = _INFO.num_cores
NS = _INFO.num_subcores
NW = NC * NS

ROWS_PER_W = N_ROWS // NW
SUB = 128
NSUB = 4
BLOCK = SUB * NSUB
N_BLOCKS = ROWS_PER_W // BLOCK

_mesh = plsc.VectorSubcoreMesh(core_axis_name="c", subcore_axis_name="s")


@functools.partial(
    pl.kernel,
    mesh=_mesh,
    out_type=jax.ShapeDtypeStruct((N_ROWS, EMBED_DIM), jnp.float32),
    compiler_params=pltpu.CompilerParams(use_tc_tiling_on_sc=False),
    scratch_types=[
        pltpu.VMEM((2, BLOCK), jnp.int32),
        pltpu.VMEM((2, BLOCK, EMBED_DIM), jnp.float32),
        pltpu.SemaphoreType.DMA,
        pltpu.SemaphoreType.DMA,
        pltpu.SemaphoreType.DMA,
    ],
)
def _gather_kernel(idx_hbm, table_hbm, out_hbm, idx_v, rows_v, gsem, osem, isem):
    wid = lax.axis_index("s") * NC + lax.axis_index("c")
    base0 = wid * ROWS_PER_W

    def fire(i, b):
        pltpu.async_copy(
            idx_hbm.at[pl.ds(base0 + i * BLOCK, BLOCK)],
            idx_v.at[b],
            isem,
        ).wait()
        for j in range(NSUB):
            pltpu.async_copy(
                table_hbm.at[idx_v.at[b, pl.ds(j * SUB, SUB)]],
                rows_v.at[b, pl.ds(j * SUB, SUB)],
                gsem,
            )

    def drain_and_store(i, b):
        for j in range(NSUB):
            pltpu.make_async_copy(
                table_hbm.at[idx_v.at[b, pl.ds(j * SUB, SUB)]],
                rows_v.at[b, pl.ds(j * SUB, SUB)],
                gsem,
            ).wait()
        pltpu.async_copy(
            rows_v.at[b],
            out_hbm.at[pl.ds(base0 + i * BLOCK, BLOCK)],
            osem,
        )

    def wait_store(i, b):
        pltpu.make_async_copy(
            rows_v.at[b],
            out_hbm.at[pl.ds(base0 + i * BLOCK, BLOCK)],
            osem,
        ).wait()

    fire(0, 0)

    def body(i, carry):
        b = lax.rem(i, 2)
        nb = 1 - b

        @pl.when(i + 1 < N_BLOCKS)
        def _():
            @pl.when(i >= 1)
            def _():
                wait_store(i + 1, nb)

            fire(i + 1, nb)

        drain_and_store(i, b)
        return carry

    lax.fori_loop(0, N_BLOCKS, body, 0)
    wait_store(N_BLOCKS - 1, (N_BLOCKS - 1) % 2)
    wait_store(N_BLOCKS - 2, (N_BLOCKS - 2) % 2)


def kernel(dist, dist_embedder_weight):
    idx_flat = dist.reshape(N_ROWS).astype(jnp.int32)
    out = _gather_kernel(idx_flat, dist_embedder_weight)
    return out.reshape(BATCH, HIST, EMBED_DIM)
